# baseline (speedup 1.0000x reference)
import functools

import jax
import jax.numpy as jnp
from jax import lax
from jax.experimental import pallas as pl
from jax.experimental.pallas import tpu as pltpu
from jax.experimental.pallas import tpu_sc as plsc

VOCAB = 1000000
D = 64
B = 4096
L = 200
BL = B * L

NC = 2
NS = 16
NW = NC * NS
ROWS_PER_W = BL // NW
CHUNK = 2 * L
NCHUNK = ROWS_PER_W // CHUNK
NGRP = CHUNK // 16
GSUB = 80
NSUB = CHUNK // GSUB
NK = D // 16


def _rsqrt(x):
    i = plsc.bitcast(x, jnp.int32)
    i = 0x5F3759DF - lax.shift_right_arithmetic(i, 1)
    y = plsc.bitcast(i, jnp.float32)
    for _ in range(3):
        y = y * (1.5 - 0.5 * x * y * y)
    return y


def _splat(v, lane):
    return jnp.take_along_axis(v, jnp.full((16,), lane, jnp.int32), axis=0)


def _sc_body(tok_hbm, table_hbm, pos_hbm, gb_hbm, bb_hbm, out_hbm,
             idx_t, rows_v, pos_v, gb_v, bb_v, sem_g0, sem_g1, sem_o):
    wid = lax.axis_index("s") * NC + lax.axis_index("c")
    pltpu.sync_copy(pos_hbm, pos_v)
    pltpu.sync_copy(gb_hbm, gb_v)
    pltpu.sync_copy(bb_hbm, bb_v)
    gvec = [gb_v[pl.ds(k * 16, 16)] for k in range(NK)]
    bvec = [bb_v[pl.ds(k * 16, 16)] for k in range(NK)]
    sems = [sem_g0, sem_g1]

    def fire_gathers(buf, sem):
        for j in range(NSUB):
            pltpu.async_copy(
                table_hbm.at[idx_t.at[buf, pl.ds(j * GSUB, GSUB)]],
                rows_v.at[buf, pl.ds(j * GSUB, GSUB)], sem)

    def drain_gathers(buf, sem):
        for j in range(NSUB):
            pltpu.make_async_copy(
                table_hbm.at[idx_t.at[buf, pl.ds(j * GSUB, GSUB)]],
                rows_v.at[buf, pl.ds(j * GSUB, GSUB)], sem).wait()

    pltpu.sync_copy(tok_hbm.at[wid * NCHUNK], idx_t.at[0])
    fire_gathers(0, sems[0])

    def chunk_body(ci, carry):
        b = lax.rem(ci, 2)
        nb = 1 - b
        out_ofs = wid * ROWS_PER_W + ci * CHUNK

        @pl.when(ci < NCHUNK - 1)
        def _prefetch():
            pltpu.sync_copy(tok_hbm.at[wid * NCHUNK + ci + 1], idx_t.at[nb])

        @pl.when(ci > 0)
        def _drain_out():
            pltpu.make_async_copy(
                rows_v.at[nb], out_hbm.at[pl.ds(out_ofs - CHUNK, CHUNK)],
                sem_o).wait()

        @pl.when(ci < NCHUNK - 1)
        def _fire_next():
            @pl.when(nb == 0)
            def _():
                fire_gathers(0, sems[0])

            @pl.when(nb == 1)
            def _():
                fire_gathers(1, sems[1])

        @pl.when(b == 0)
        def _():
            drain_gathers(0, sems[0])

        @pl.when(b == 1)
        def _():
            drain_gathers(1, sems[1])

        def group_body(g, gcarry):
            tok_v = idx_t[b, pl.ds(g * 16, 16)]
            maskf = jnp.where(tok_v != 0, 1.0, 0.0).astype(jnp.float32)
            for r in range(16):
                row = g * 16 + r
                s = [rows_v[b, row, pl.ds(k * 16, 16)] +
                     pos_v[row, pl.ds(k * 16, 16)] for k in range(NK)]
                part = (s[0] + s[1]) + (s[2] + s[3])
                tot = _splat(plsc.cumsum(part), 15)
                sq = (s[0] * s[0] + s[1] * s[1]) + (s[2] * s[2] + s[3] * s[3])
                tot2 = _splat(plsc.cumsum(sq), 15)
                mu = tot * (1.0 / D)
                var = tot2 * (1.0 / D) - mu * mu + 1e-5
                rs = _rsqrt(var)
                m = _splat(maskf, r)
                a = rs * m
                bco = (0.0 - mu * rs) * m
                for k in range(NK):
                    o = (s[k] * a + bco) * gvec[k] + bvec[k] * m
                    rows_v[b, row, pl.ds(k * 16, 16)] = o
            return gcarry

        lax.fori_loop(0, NGRP, group_body, 0)
        pltpu.async_copy(rows_v.at[b], out_hbm.at[pl.ds(out_ofs, CHUNK)],
                         sem_o)
        return carry

    lax.fori_loop(0, NCHUNK, chunk_body, 0)
    pltpu.make_async_copy(
        rows_v.at[1], out_hbm.at[pl.ds(wid * ROWS_PER_W +
                                       (NCHUNK - 1) * CHUNK, CHUNK)],
        sem_o).wait()


def kernel(tokens, tok_table, pos_table, gamma, beta):
    tok3 = tokens.reshape(BL // CHUNK, CHUNK).astype(jnp.int32)
    pos_lin = jnp.tile(pos_table, (CHUNK // L, 1))

    sc = functools.partial(
        pl.kernel,
        mesh=plsc.VectorSubcoreMesh(core_axis_name="c", subcore_axis_name="s"),
        out_type=jax.ShapeDtypeStruct((BL, D), jnp.float32),
        compiler_params=pltpu.CompilerParams(needs_layout_passes=False,
                                             use_tc_tiling_on_sc=False),
        scratch_types=[
            pltpu.VMEM((2, CHUNK), jnp.int32),
            pltpu.VMEM((2, CHUNK, D), jnp.float32),
            pltpu.VMEM((CHUNK, D), jnp.float32),
            pltpu.VMEM((D,), jnp.float32),
            pltpu.VMEM((D,), jnp.float32),
            pltpu.SemaphoreType.DMA,
            pltpu.SemaphoreType.DMA,
            pltpu.SemaphoreType.DMA,
        ],
    )(_sc_body)
    out = sc(tok3, tok_table, pos_lin, gamma, beta)
    return out.reshape(B, L, D)


# speedup vs baseline: 1.5347x; 1.5347x over previous
import functools

import jax
import jax.numpy as jnp
from jax import lax
from jax.experimental import pallas as pl
from jax.experimental.pallas import tpu as pltpu
from jax.experimental.pallas import tpu_sc as plsc

VOCAB = 1000000
D = 64
B = 4096
L = 200
BL = B * L

NC = 2
NS = 16
NW = NC * NS
ROWS_PER_W = BL // NW
CHUNK = 2 * L
NCHUNK = ROWS_PER_W // CHUNK
NGRP = CHUNK // 16
GSUB = 80
NSUB = CHUNK // GSUB
NK = D // 16


def _rsqrt(x):
    i = plsc.bitcast(x, jnp.int32)
    i = 0x5F3759DF - lax.shift_right_arithmetic(i, 1)
    y = plsc.bitcast(i, jnp.float32)
    for _ in range(2):
        y = y * (1.5 - 0.5 * x * y * y)
    return y


def _splat(v, lane):
    return jnp.take_along_axis(v, jnp.full((16,), lane, jnp.int32), axis=0)


def _sc_body(tok_hbm, table_hbm, pos_hbm, gb_hbm, bb_hbm, out_hbm,
             idx_t, rows_v, pos_v, gb_v, bb_v, sem_g0, sem_g1, sem_o):
    wid = lax.axis_index("s") * NC + lax.axis_index("c")
    pltpu.sync_copy(pos_hbm, pos_v)
    pltpu.sync_copy(gb_hbm, gb_v)
    pltpu.sync_copy(bb_hbm, bb_v)
    gvec = [gb_v[pl.ds(k * 16, 16)] for k in range(NK)]
    bvec = [bb_v[pl.ds(k * 16, 16)] for k in range(NK)]
    sems = [sem_g0, sem_g1]

    def fire_gathers(buf, sem):
        for j in range(NSUB):
            pltpu.async_copy(
                table_hbm.at[idx_t.at[buf, pl.ds(j * GSUB, GSUB)]],
                rows_v.at[buf, pl.ds(j * GSUB, GSUB)], sem)

    def drain_gathers(buf, sem):
        for j in range(NSUB):
            pltpu.make_async_copy(
                table_hbm.at[idx_t.at[buf, pl.ds(j * GSUB, GSUB)]],
                rows_v.at[buf, pl.ds(j * GSUB, GSUB)], sem).wait()

    pltpu.sync_copy(tok_hbm.at[wid * NCHUNK], idx_t.at[0])
    fire_gathers(0, sems[0])

    def compute_chunk(b, ci):
        def group_body(g, gcarry):
            tok_v = idx_t[b, pl.ds(g * 16, 16)]
            maskf = jnp.where(tok_v != 0, 1.0, 0.0).astype(jnp.float32)
            for r in range(16):
                row = g * 16 + r
                s = [rows_v[b, row, pl.ds(k * 16, 16)] +
                     pos_v[row, pl.ds(k * 16, 16)] for k in range(NK)]
                part = (s[0] + s[1]) + (s[2] + s[3])
                tot = _splat(plsc.cumsum(part), 15)
                sq = (s[0] * s[0] + s[1] * s[1]) + (s[2] * s[2] + s[3] * s[3])
                tot2 = _splat(plsc.cumsum(sq), 15)
                mu = tot * (1.0 / D)
                var = tot2 * (1.0 / D) - mu * mu + 1e-5
                rs = _rsqrt(var)
                m = _splat(maskf, r)
                a = rs * m
                bco = (0.0 - mu * rs) * m
                for k in range(NK):
                    o = (s[k] * a + bco) * gvec[k] + bvec[k] * m
                    rows_v[b, row, pl.ds(k * 16, 16)] = o
            return gcarry

        lax.fori_loop(0, NGRP, group_body, 0)

    def half_body(b, ci):
        nb = 1 - b
        out_ofs = wid * ROWS_PER_W + ci * CHUNK

        @pl.when(ci < NCHUNK - 1)
        def _prefetch():
            pltpu.sync_copy(tok_hbm.at[wid * NCHUNK + ci + 1], idx_t.at[nb])

        @pl.when(ci > 0)
        def _drain_out():
            pltpu.make_async_copy(
                rows_v.at[nb], out_hbm.at[pl.ds(out_ofs - CHUNK, CHUNK)],
                sem_o).wait()

        @pl.when(ci < NCHUNK - 1)
        def _fire_next():
            fire_gathers(nb, sems[nb])

        drain_gathers(b, sems[b])
        compute_chunk(b, ci)
        pltpu.async_copy(rows_v.at[b], out_hbm.at[pl.ds(out_ofs, CHUNK)],
                         sem_o)

    def pair_body(cp, carry):
        half_body(0, cp * 2)
        half_body(1, cp * 2 + 1)
        return carry

    lax.fori_loop(0, NCHUNK // 2, pair_body, 0)
    pltpu.make_async_copy(
        rows_v.at[1], out_hbm.at[pl.ds(wid * ROWS_PER_W +
                                       (NCHUNK - 1) * CHUNK, CHUNK)],
        sem_o).wait()


def kernel(tokens, tok_table, pos_table, gamma, beta):
    tok3 = tokens.reshape(BL // CHUNK, CHUNK).astype(jnp.int32)
    pos_lin = jnp.tile(pos_table, (CHUNK // L, 1))

    sc = functools.partial(
        pl.kernel,
        mesh=plsc.VectorSubcoreMesh(core_axis_name="c", subcore_axis_name="s"),
        out_type=jax.ShapeDtypeStruct((BL, D), jnp.float32),
        compiler_params=pltpu.CompilerParams(needs_layout_passes=False,
                                             use_tc_tiling_on_sc=False),
        scratch_types=[
            pltpu.VMEM((2, CHUNK), jnp.int32),
            pltpu.VMEM((2, CHUNK, D), jnp.float32),
            pltpu.VMEM((CHUNK, D), jnp.float32),
            pltpu.VMEM((D,), jnp.float32),
            pltpu.VMEM((D,), jnp.float32),
            pltpu.SemaphoreType.DMA,
            pltpu.SemaphoreType.DMA,
            pltpu.SemaphoreType.DMA,
        ],
    )(_sc_body)
    out = sc(tok3, tok_table, pos_lin, gamma, beta)
    return out.reshape(B, L, D)
